# baseline (speedup 1.0000x reference)
import functools

import jax
import jax.numpy as jnp
from jax import lax
from jax.experimental import pallas as pl
from jax.experimental.pallas import tpu as pltpu
from jax.experimental.pallas import tpu_sc as plsc

Q, C, D = 4096, 32, 64
NC, NS, L = 2, 16, 16
NW = NC * NS
QPW = Q // NW
NV = D // L
INVERSE_SIGMA = 10.0


def _rsqrt(p2):
    i = plsc.bitcast(p2, jnp.int32)
    y = plsc.bitcast(jnp.int32(0x5F3759DF) - (i >> 1), jnp.float32)
    hx = 0.5 * p2
    for _ in range(3):
        y = y * (1.5 - hx * y * y)
    return y


def _expm1(z):
    return jnp.where(jnp.abs(z) < 1e-3, z + 0.5 * z * z, jnp.exp(z) - 1.0)


_mesh = plsc.VectorSubcoreMesh(core_axis_name="c", subcore_axis_name="s")


@functools.partial(
    pl.kernel,
    mesh=_mesh,
    compiler_params=pltpu.CompilerParams(
        needs_layout_passes=False, use_tc_tiling_on_sc=False),
    out_type=jax.ShapeDtypeStruct((Q, D), jnp.float32),
    scratch_types=[
        pltpu.VMEM((QPW, D), jnp.float32),
        pltpu.VMEM((QPW, D), jnp.float32),
        pltpu.VMEM((QPW, C), jnp.int32),
        pltpu.VMEM((2, C, D), jnp.float32),
        pltpu.VMEM((QPW, D), jnp.float32),
        pltpu.SemaphoreType.DMA,
        pltpu.SemaphoreType.DMA,
    ],
)
def _nc_kernel(x_hbm, v_hbm, k_hbm, X_hbm, out_hbm,
               x_v, v_v, k_v, rows_v, out_v, sem0, sem1):
    wid = lax.axis_index("s") * NC + lax.axis_index("c")
    base = wid * QPW
    pltpu.sync_copy(x_hbm.at[pl.ds(base, QPW)], x_v)
    pltpu.sync_copy(v_hbm.at[pl.ds(base, QPW)], v_v)
    pltpu.sync_copy(k_hbm.at[pl.ds(base, QPW)], k_v)

    lane = lax.broadcasted_iota(jnp.int32, (L,), 0)
    zero = jnp.zeros((L,), jnp.float32)
    cidx = [lane + (g * L) for g in range(2)]
    jidx = [lane * 0 + j for j in range(D)]

    def _sum16(vec):
        e = [vec[i] for i in range(L)]
        while len(e) > 1:
            e = [e[i] + e[i + 1] for i in range(0, len(e), 2)]
        return e[0]

    def compute(q, buf):
        xq = [x_v[q, pl.ds(L * i, L)] for i in range(NV)]
        vq = [v_v[q, pl.ds(L * i, L)] for i in range(NV)]
        vv = vq[0] * vq[0]
        vxv = vq[0] * xq[0]
        xxv = xq[0] * xq[0]
        for i in range(1, NV):
            vv = vv + vq[i] * vq[i]
            vxv = vxv + vq[i] * xq[i]
            xxv = xxv + xq[i] * xq[i]
        nv2 = _sum16(vv)
        vx = _sum16(vxv)
        xx = _sum16(xxv)

        dotv, n2v = [], []
        for g in range(2):
            a_vr, a_rr, a_xr = zero, zero, zero
            for j in range(D):
                rj = plsc.load_gather(buf, [cidx[g], jidx[j]])
                vj = vq[j // L][j % L]
                xj = xq[j // L][j % L]
                a_vr = a_vr + vj * rj
                a_rr = a_rr + rj * rj
                a_xr = a_xr + xj * rj
            dotv.append(a_vr - vx)
            n2v.append(a_rr - 2.0 * a_xr + xx)

        tg = []
        for g in range(2):
            p2 = n2v[g] * nv2
            denom = jnp.maximum(p2 * _rsqrt(p2), 1e-8)
            tg.append(_expm1(INVERSE_SIGMA * (dotv[g] / denom)))
        sum_t = _sum16(tg[0] + tg[1])
        sum_abs = _sum16(jnp.abs(tg[0]) + jnp.abs(tg[1]))
        mean_t = sum_t * (1.0 / C)
        inv_s = (1.0 + zero) / (sum_abs + zero)
        wv = [(tg[g] - mean_t) * inv_s for g in range(2)]

        o = [zero] * NV
        for c in range(C):
            g, m = divmod(c, L)
            wc = wv[g][m]
            for i in range(NV):
                o[i] = o[i] + wc * (buf[c, pl.ds(L * i, L)] - xq[i])
        for i in range(NV):
            out_v[q, pl.ds(L * i, L)] = o[i]

    def step(q, _):
        pltpu.async_copy(X_hbm.at[k_v.at[q]], rows_v.at[0], sem0).wait()
        compute(q, rows_v.at[0])
        return _

    lax.fori_loop(0, QPW, step, 0, unroll=False)
    pltpu.sync_copy(out_v, out_hbm.at[pl.ds(base, QPW)])


def kernel(x, v, k, X):
    return _nc_kernel(x, v, k.astype(jnp.int32), X)


# speedup vs baseline: 1.2017x; 1.2017x over previous
import functools

import jax
import jax.numpy as jnp
from jax import lax
from jax.experimental import pallas as pl
from jax.experimental.pallas import tpu as pltpu
from jax.experimental.pallas import tpu_sc as plsc

Q, C, D = 4096, 32, 64
NC, NS, L = 2, 16, 16
NW = NC * NS
QPW = Q // NW
QPB = 4
NB = QPW // QPB
NV = D // L
INVERSE_SIGMA = 10.0


def _rsqrt(p2):
    i = plsc.bitcast(p2, jnp.int32)
    y = plsc.bitcast(jnp.int32(0x5F3759DF) - (i >> 1), jnp.float32)
    hx = 0.5 * p2
    for _ in range(3):
        y = y * (1.5 - hx * y * y)
    return y


def _expm1(z):
    return jnp.where(jnp.abs(z) < 1e-3, z + 0.5 * z * z, jnp.exp(z) - 1.0)


_mesh = plsc.VectorSubcoreMesh(core_axis_name="c", subcore_axis_name="s")


@functools.partial(
    pl.kernel,
    mesh=_mesh,
    compiler_params=pltpu.CompilerParams(
        needs_layout_passes=False, use_tc_tiling_on_sc=False),
    out_type=jax.ShapeDtypeStruct((Q, D), jnp.float32),
    scratch_types=[
        pltpu.VMEM((QPW, D), jnp.float32),
        pltpu.VMEM((QPW, D), jnp.float32),
        pltpu.VMEM((NB, QPB * C), jnp.int32),
        pltpu.VMEM((2, QPB * C, D), jnp.float32),
        pltpu.VMEM((QPW, D), jnp.float32),
        pltpu.SemaphoreType.DMA((2,)),
    ],
)
def _nc_kernel(x_hbm, v_hbm, k_hbm, X_hbm, out_hbm,
               x_v, v_v, k_v, rows_v, out_v, sem):
    wid = lax.axis_index("s") * NC + lax.axis_index("c")
    base = wid * QPW
    pltpu.sync_copy(x_hbm.at[pl.ds(base, QPW)], x_v)
    pltpu.sync_copy(v_hbm.at[pl.ds(base, QPW)], v_v)
    pltpu.sync_copy(k_hbm.at[pl.ds(wid * NB, NB)], k_v)

    lane = lax.broadcasted_iota(jnp.int32, (L,), 0)
    zero = jnp.zeros((L,), jnp.float32)
    cidx = [[lane + (u * C + g * L) for g in range(2)] for u in range(QPB)]
    jidx = [lane * 0 + j for j in range(D)]

    def _sum16(vec):
        e = [vec[i] for i in range(L)]
        while len(e) > 1:
            e = [e[i] + e[i + 1] for i in range(0, len(e), 2)]
        return e[0]

    def compute(q, buf, u):
        xq = [x_v[q, pl.ds(L * i, L)] for i in range(NV)]
        vq = [v_v[q, pl.ds(L * i, L)] for i in range(NV)]
        vv = vq[0] * vq[0]
        vxv = vq[0] * xq[0]
        xxv = xq[0] * xq[0]
        for i in range(1, NV):
            vv = vv + vq[i] * vq[i]
            vxv = vxv + vq[i] * xq[i]
            xxv = xxv + xq[i] * xq[i]
        nv2 = _sum16(vv)
        vx = _sum16(vxv)
        xx = _sum16(xxv)
        vs = [vq[j // L][j % L] for j in range(D)]
        xs = [xq[j // L][j % L] for j in range(D)]

        dotv, n2v = [], []
        for g in range(2):
            a_vr, a_rr, a_xr = zero, zero, zero
            for j in range(D):
                rj = plsc.load_gather(buf, [cidx[u][g], jidx[j]])
                a_vr = a_vr + vs[j] * rj
                a_rr = a_rr + rj * rj
                a_xr = a_xr + xs[j] * rj
            dotv.append(a_vr - vx)
            n2v.append(a_rr - 2.0 * a_xr + xx)

        tg = []
        for g in range(2):
            p2 = n2v[g] * nv2
            denom = jnp.maximum(p2 * _rsqrt(p2), 1e-8)
            tg.append(_expm1(INVERSE_SIGMA * (dotv[g] / denom)))
        sum_t = _sum16(tg[0] + tg[1])
        sum_abs = _sum16(jnp.abs(tg[0]) + jnp.abs(tg[1]))
        mean_t = sum_t * (1.0 / C)
        inv_s = (1.0 + zero) / (sum_abs + zero)
        wv = [(tg[g] - mean_t) * inv_s for g in range(2)]

        o = [zero] * NV
        for c in range(C):
            g, m = divmod(c, L)
            wc = wv[g][m]
            for i in range(NV):
                o[i] = o[i] + wc * buf[u * C + c, pl.ds(L * i, L)]
        for i in range(NV):
            out_v[q, pl.ds(L * i, L)] = o[i]

    def gather(b, slot):
        return pltpu.make_async_copy(
            X_hbm.at[k_v.at[b]], rows_v.at[slot], sem.at[slot])

    gather(0, 0).start()

    def body(i, carry):
        slot = lax.rem(i, 2)
        nxt = lax.rem(i + 1, 2)
        gather(jnp.minimum(i + 1, NB - 1), nxt).start()
        gather(i, slot).wait()
        buf = rows_v.at[slot]
        for u in range(QPB):
            compute(i * QPB + u, buf, u)
        return carry

    lax.fori_loop(0, NB, body, 0, unroll=False)
    gather(NB - 1, lax.rem(NB, 2)).wait()
    pltpu.sync_copy(out_v, out_hbm.at[pl.ds(base, QPW)])


def kernel(x, v, k, X):
    k32 = k.astype(jnp.int32).reshape(Q // QPB, QPB * C)
    return _nc_kernel(x, v, k32, X)


# speedup vs baseline: 1.2544x; 1.0439x over previous
import functools

import jax
import jax.numpy as jnp
from jax import lax
from jax.experimental import pallas as pl
from jax.experimental.pallas import tpu as pltpu
from jax.experimental.pallas import tpu_sc as plsc

Q, C, D = 4096, 32, 64
NC, NS, L = 2, 16, 16
NW = NC * NS
QPW = Q // NW
QPB = 4
NB = QPW // QPB
NV = D // L
INVERSE_SIGMA = 10.0


def _rsqrt(p2):
    i = plsc.bitcast(p2, jnp.int32)
    y = plsc.bitcast(jnp.int32(0x5F3759DF) - (i >> 1), jnp.float32)
    hx = 0.5 * p2
    for _ in range(3):
        y = y * (1.5 - hx * y * y)
    return y


def _expm1(z):
    return jnp.where(jnp.abs(z) < 1e-3, z + 0.5 * z * z, jnp.exp(z) - 1.0)


_mesh = plsc.VectorSubcoreMesh(core_axis_name="c", subcore_axis_name="s")


@functools.partial(
    pl.kernel,
    mesh=_mesh,
    compiler_params=pltpu.CompilerParams(
        needs_layout_passes=False, use_tc_tiling_on_sc=False),
    out_type=jax.ShapeDtypeStruct((Q, D), jnp.float32),
    scratch_types=[
        pltpu.VMEM((QPW, D), jnp.float32),
        pltpu.VMEM((QPW, D), jnp.float32),
        pltpu.VMEM((NB, QPB * C), jnp.int32),
        pltpu.VMEM((2, QPB * C, D), jnp.float32),
        pltpu.VMEM((QPW, D), jnp.float32),
        pltpu.VMEM((L,), jnp.float32),
        pltpu.SemaphoreType.DMA((2,)),
    ],
)
def _nc_kernel(x_hbm, v_hbm, k_hbm, X_hbm, out_hbm,
               x_v, v_v, k_v, rows_v, out_v, red_v, sem):
    wid = lax.axis_index("s") * NC + lax.axis_index("c")
    base = wid * QPW
    pltpu.sync_copy(x_hbm.at[pl.ds(base, QPW)], x_v)
    pltpu.sync_copy(v_hbm.at[pl.ds(base, QPW)], v_v)
    pltpu.sync_copy(k_hbm.at[pl.ds(wid * NB, NB)], k_v)

    lane = lax.broadcasted_iota(jnp.int32, (L,), 0)
    zero = jnp.zeros((L,), jnp.float32)
    cidx = [[lane + (u * C + g * L) for g in range(2)] for u in range(QPB)]
    jidx = [lane * 0 + j for j in range(D)]

    shuf = [lane ^ sh for sh in (8, 4, 2, 1)]

    def _allsum16(vec):
        for sidx in shuf:
            red_v[...] = vec
            vec = vec + plsc.load_gather(red_v, [sidx])
        return vec

    def compute(q, buf, u):
        xq = [x_v[q, pl.ds(L * i, L)] for i in range(NV)]
        vq = [v_v[q, pl.ds(L * i, L)] for i in range(NV)]
        vv = (vq[0] * vq[0] + vq[1] * vq[1]) + (vq[2] * vq[2] + vq[3] * vq[3])
        nv2 = _allsum16(vv)
        vs = [vq[j // L][j % L] for j in range(D)]
        xs = [xq[j // L][j % L] for j in range(D)]

        dotv, n2v = [], []
        for g in range(2):
            a = [zero] * 4
            for j in range(D):
                rj = plsc.load_gather(buf, [cidx[u][g], jidx[j]])
                d = rj - xs[j]
                p = j & 1
                a[p] = a[p] + vs[j] * d
                a[2 + p] = a[2 + p] + d * d
            dotv.append(a[0] + a[1])
            n2v.append(a[2] + a[3])

        tg = []
        for g in range(2):
            p2 = n2v[g] * nv2
            denom = jnp.maximum(p2 * _rsqrt(p2), 1e-8)
            tg.append(_expm1(INVERSE_SIGMA * (dotv[g] / denom)))
        sum_t = _allsum16(tg[0] + tg[1])
        sum_abs = _allsum16(jnp.abs(tg[0]) + jnp.abs(tg[1]))
        mean_t = sum_t * (1.0 / C)
        inv_s = (1.0 + zero) / sum_abs
        wv = [(tg[g] - mean_t) * inv_s for g in range(2)]

        o = [zero] * (2 * NV)
        for c in range(C):
            g, m = divmod(c, L)
            wc = wv[g][m]
            p = c & 1
            for i in range(NV):
                oi = 2 * i + p
                o[oi] = o[oi] + wc * buf[u * C + c, pl.ds(L * i, L)]
        for i in range(NV):
            out_v[q, pl.ds(L * i, L)] = o[2 * i] + o[2 * i + 1]

    def gather(b, slot):
        return pltpu.make_async_copy(
            X_hbm.at[k_v.at[b]], rows_v.at[slot], sem.at[slot])

    gather(0, 0).start()

    def body(i, carry):
        slot = lax.rem(i, 2)
        nxt = lax.rem(i + 1, 2)
        gather(jnp.minimum(i + 1, NB - 1), nxt).start()
        gather(i, slot).wait()
        buf = rows_v.at[slot]
        for u in range(QPB):
            compute(i * QPB + u, buf, u)
        return carry

    lax.fori_loop(0, NB, body, 0, unroll=False)
    gather(NB - 1, lax.rem(NB, 2)).wait()
    pltpu.sync_copy(out_v, out_hbm.at[pl.ds(base, QPW)])


def kernel(x, v, k, X):
    k32 = k.astype(jnp.int32).reshape(Q // QPB, QPB * C)
    return _nc_kernel(x, v, k32, X)


# speedup vs baseline: 1.3466x; 1.0735x over previous
import functools

import jax
import jax.numpy as jnp
from jax import lax
from jax.experimental import pallas as pl
from jax.experimental.pallas import tpu as pltpu
from jax.experimental.pallas import tpu_sc as plsc

Q, C, D = 4096, 32, 64
NC, NS, L = 2, 16, 16
NW = NC * NS
QPW = Q // NW
QPB = 4
NB = QPW // QPB
NV = D // L
INVERSE_SIGMA = 10.0


def _rsqrt(p2):
    i = plsc.bitcast(p2, jnp.int32)
    y = plsc.bitcast(jnp.int32(0x5F3759DF) - (i >> 1), jnp.float32)
    hx = 0.5 * p2
    for _ in range(3):
        y = y * (1.5 - hx * y * y)
    return y


def _expm1(z):
    return jnp.where(jnp.abs(z) < 1e-3, z + 0.5 * z * z, jnp.exp(z) - 1.0)


_mesh = plsc.VectorSubcoreMesh(core_axis_name="c", subcore_axis_name="s")


@functools.partial(
    pl.kernel,
    mesh=_mesh,
    compiler_params=pltpu.CompilerParams(
        needs_layout_passes=False, use_tc_tiling_on_sc=False),
    out_type=jax.ShapeDtypeStruct((Q, D), jnp.float32),
    scratch_types=[
        pltpu.VMEM((QPW, D), jnp.float32),
        pltpu.VMEM((QPW, D), jnp.float32),
        pltpu.VMEM((NB, QPB * C), jnp.int32),
        pltpu.VMEM((2, QPB * C, D), jnp.float32),
        pltpu.VMEM((QPW, D), jnp.float32),
        pltpu.VMEM((L,), jnp.float32),
        pltpu.SemaphoreType.DMA((2,)),
    ],
)
def _nc_kernel(x_hbm, v_hbm, k_hbm, X_hbm, out_hbm,
               x_v, v_v, k_v, rows_v, out_v, red_v, sem):
    wid = lax.axis_index("s") * NC + lax.axis_index("c")
    base = wid * QPW
    pltpu.sync_copy(x_hbm.at[pl.ds(base, QPW)], x_v)
    pltpu.sync_copy(v_hbm.at[pl.ds(base, QPW)], v_v)
    pltpu.sync_copy(k_hbm.at[pl.ds(wid * NB, NB)], k_v)

    lane = lax.broadcasted_iota(jnp.int32, (L,), 0)
    zero = jnp.zeros((L,), jnp.float32)
    cidx = [[lane + (u * C + g * L) for g in range(2)] for u in range(QPB)]
    jidx = [lane * 0 + j for j in range(D)]

    shuf = [lane ^ sh for sh in (8, 4, 2, 1)]

    def _allsum16(vec):
        for sidx in shuf:
            red_v[...] = vec
            vec = vec + plsc.load_gather(red_v, [sidx])
        return vec

    def compute(q, buf, u):
        xq = [x_v[q, pl.ds(L * i, L)] for i in range(NV)]
        vq = [v_v[q, pl.ds(L * i, L)] for i in range(NV)]
        vv = (vq[0] * vq[0] + vq[1] * vq[1]) + (vq[2] * vq[2] + vq[3] * vq[3])
        nv2 = _allsum16(vv)
        vs = [vq[j // L][j % L] for j in range(D)]
        xs = [xq[j // L][j % L] for j in range(D)]

        a = [[zero] * 4 for _ in range(2)]
        for j in range(D):
            xb, vb = xs[j], vs[j]
            p = j & 1
            for g in range(2):
                rj = plsc.load_gather(buf, [cidx[u][g], jidx[j]])
                d = rj - xb
                a[g][p] = a[g][p] + vb * d
                a[g][2 + p] = a[g][2 + p] + d * d
        dotv = [a[g][0] + a[g][1] for g in range(2)]
        n2v = [a[g][2] + a[g][3] for g in range(2)]

        tg = []
        for g in range(2):
            p2 = n2v[g] * nv2
            denom = jnp.maximum(p2 * _rsqrt(p2), 1e-8)
            tg.append(_expm1(INVERSE_SIGMA * (dotv[g] / denom)))
        sum_t = _allsum16(tg[0] + tg[1])
        sum_abs = _allsum16(jnp.abs(tg[0]) + jnp.abs(tg[1]))
        mean_t = sum_t * (1.0 / C)
        inv_s = (1.0 + zero) / sum_abs
        wv = [(tg[g] - mean_t) * inv_s for g in range(2)]

        o = [zero] * (2 * NV)
        for c in range(C):
            g, m = divmod(c, L)
            wc = wv[g][m]
            p = c & 1
            for i in range(NV):
                oi = 2 * i + p
                o[oi] = o[oi] + wc * buf[u * C + c, pl.ds(L * i, L)]
        for i in range(NV):
            out_v[q, pl.ds(L * i, L)] = o[2 * i] + o[2 * i + 1]

    def gather(b, slot):
        return pltpu.make_async_copy(
            X_hbm.at[k_v.at[b]], rows_v.at[slot], sem.at[slot])

    gather(0, 0).start()

    def body(i, carry):
        slot = lax.rem(i, 2)
        nxt = lax.rem(i + 1, 2)
        gather(jnp.minimum(i + 1, NB - 1), nxt).start()
        gather(i, slot).wait()
        buf = rows_v.at[slot]
        for u in range(QPB):
            compute(i * QPB + u, buf, u)
        return carry

    lax.fori_loop(0, NB, body, 0, unroll=False)
    gather(NB - 1, lax.rem(NB, 2)).wait()
    pltpu.sync_copy(out_v, out_hbm.at[pl.ds(base, QPW)])


def kernel(x, v, k, X):
    k32 = k.astype(jnp.int32).reshape(Q // QPB, QPB * C)
    return _nc_kernel(x, v, k32, X)
